# baseline (speedup 1.0000x reference)
import functools

import jax
import jax.numpy as jnp
import numpy as np
from jax import lax
from jax.experimental import pallas as pl
from jax.experimental.pallas import tpu as pltpu
from jax.experimental.pallas import tpu_sc as plsc

F32 = jnp.float32
_LOCS = (0, 16, 32, 48, 64, 80, 96, 111)


def _sc_pool():
    mesh = plsc.VectorSubcoreMesh(core_axis_name="c", subcore_axis_name="s")
    out_type = tuple(
        jax.ShapeDtypeStruct((512, 128), F32) for _ in range(8)
    )
    scratch = [pltpu.VMEM((24, 128), F32) for _ in range(12)] + [
        pltpu.VMEM((16, 128), F32) for _ in range(8)
    ]

    @functools.partial(
        pl.kernel, out_type=out_type, mesh=mesh, scratch_types=scratch
    )
    def k(ael, aer, aol, aor, pel, per, pol, por, nel, ner, nol, nor,
          out_pel, out_per, out_pol, out_por,
          out_nel, out_ner, out_nol, out_nor,
          ael_v, aer_v, aol_v, aor_v, pel_v, per_v, pol_v, por_v,
          nel_v, ner_v, nol_v, nor_v,
          ebl_p, ebr_p, obl_p, obr_p, ebl_n, ebr_n, obl_n, obr_n):
        wid = lax.axis_index("c") * 16 + lax.axis_index("s")
        r_top = 16 * wid
        ji = lax.iota(jnp.int32, 16)
        for src, dst in (
            (ael, ael_v), (aer, aer_v), (aol, aol_v), (aor, aor_v),
            (pel, pel_v), (per, per_v), (pol, pol_v), (por, por_v),
            (nel, nel_v), (ner, ner_v), (nol, nol_v), (nor, nor_v),
        ):
            pltpu.sync_copy(src.at[pl.ds(r_top, 24)], dst)

        def row_body(t, carry):
            lr = 2 * t
            for aeh, aoh, sel in (
                (ael_v, aol_v,
                 ((pel_v, pol_v, ebl_p, obl_p), (nel_v, nol_v, ebl_n, obl_n))),
                (aer_v, aor_v,
                 ((per_v, por_v, ebr_p, obr_p), (ner_v, nor_v, ebr_n, obr_n))),
            ):
                for l0 in _LOCS:
                    a_t = []
                    for r in range(3):
                        a_t.append(aeh[lr + r, pl.ds(l0, 16)])
                        a_t.append(aoh[lr + r, pl.ds(l0, 16)])
                        a_t.append(aeh[lr + r, pl.ds(l0 + 1, 16)])
                    for ceh, coh, eblk, oblk in sel:
                        c_t = []
                        for r in range(3):
                            c_t.append(ceh[lr + r, pl.ds(l0, 16)])
                            c_t.append(coh[lr + r, pl.ds(l0, 16)])
                            c_t.append(ceh[lr + r, pl.ds(l0 + 1, 16)])
                        is_min = eblk is ebl_p or eblk is ebr_p
                        bd = jnp.abs(a_t[0] - c_t[0])
                        bv = c_t[0]
                        for kk in range(1, 9):
                            dk = jnp.abs(a_t[kk] - c_t[kk])
                            m = (dk < bd) if is_min else (dk > bd)
                            bv = jnp.where(m, c_t[kk], bv)
                            bd = jnp.where(m, dk, bd)
                        eblk[lr, pl.ds(l0, 16)] = bv
                        eblk[lr + 1, pl.ds(l0, 16)] = bv
                        oblk[lr, pl.ds(l0, 16)] = bv
                        oblk[lr + 1, pl.ds(l0, 16)] = bv
            a127 = []
            p127 = []
            n127 = []
            for r in range(3):
                for taps, el_v, ol_v, er_v in (
                    (a127, ael_v, aol_v, aer_v),
                    (p127, pel_v, pol_v, per_v),
                    (n127, nel_v, nol_v, ner_v),
                ):
                    taps.append(el_v[lr + r, pl.ds(112, 16)][15])
                    taps.append(ol_v[lr + r, pl.ds(112, 16)][15])
                    taps.append(er_v[lr + r, pl.ds(0, 16)][0])
            for c127, is_min, eblk, oblk in (
                (p127, True, ebl_p, obl_p),
                (n127, False, ebl_n, obl_n),
            ):
                bd = jnp.abs(a127[0] - c127[0])
                bv = c127[0]
                for kk in range(1, 9):
                    dk = jnp.abs(a127[kk] - c127[kk])
                    m = (dk < bd) if is_min else (dk > bd)
                    bv = jnp.where(m, c127[kk], bv)
                    bd = jnp.where(m, dk, bd)
                for blk in (eblk, oblk):
                    for br in (lr, lr + 1):
                        v = blk[br, pl.ds(112, 16)]
                        blk[br, pl.ds(112, 16)] = jnp.where(ji == 15, bv, v)
            for coh, ebr, obr in ((por_v, ebr_p, obr_p), (nor_v, ebr_n, obr_n)):
                tail = ebr[lr, pl.ds(112, 16)]
                ev = jnp.where(ji == 15, tail[14], tail)
                ebr[lr, pl.ds(112, 16)] = ev
                ebr[lr + 1, pl.ds(112, 16)] = ev
                c0 = coh[lr, pl.ds(112, 16)]
                c1 = coh[lr + 1, pl.ds(112, 16)]
                obr[lr, pl.ds(112, 16)] = jnp.where(ji == 15, c0[15], tail)
                obr[lr + 1, pl.ds(112, 16)] = jnp.where(ji == 15, c1[15], tail)
            return carry

        for _t in range(8):
            row_body(_t, 0)

        @pl.when(wid == 31)
        def _tail():
            for ceh, coh, celh, colh, ebl, obl, ebr, obr in (
                (per_v, por_v, pel_v, pol_v, ebl_p, obl_p, ebr_p, obr_p),
                (ner_v, nor_v, nel_v, nol_v, ebl_n, obl_n, ebr_n, obr_n),
            ):
                for blk, src15 in (
                    (ebl, celh), (obl, colh), (ebr, ceh), (obr, coh),
                ):
                    for tt in range(8):
                        blk[14, pl.ds(16 * tt, 16)] = blk[13, pl.ds(16 * tt, 16)]
                        blk[15, pl.ds(16 * tt, 16)] = src15[15, pl.ds(16 * tt, 16)]
                t14 = obr[14, pl.ds(112, 16)]
                c2 = coh[14, pl.ds(112, 16)]
                obr[14, pl.ds(112, 16)] = jnp.where(ji == 15, c2[15], t14)

        for blk, out in (
            (ebl_p, out_pel), (ebr_p, out_per), (obl_p, out_pol),
            (obr_p, out_por), (ebl_n, out_nel), (ebr_n, out_ner),
            (obl_n, out_nol), (obr_n, out_nor),
        ):
            pltpu.sync_copy(blk, out.at[pl.ds(r_top, 16)])

    return k


_POOL = _sc_pool()

_ME = np.zeros((512, 256), np.float32)
_ME[2 * np.arange(256), np.arange(256)] = 1.0
_MO = np.zeros((512, 256), np.float32)
_MO[2 * np.arange(256) + 1, np.arange(256)] = 1.0


BF16 = jnp.bfloat16


def _dot1(x, y):
    return lax.dot_general(
        x, y, (((1,), (0,)), ((), ())), preferred_element_type=F32,
    )


def _split3(x):
    hi = x.astype(BF16)
    r1 = x - hi.astype(F32)
    mid = r1.astype(BF16)
    lo = (r1 - mid.astype(F32)).astype(BF16)
    return hi, mid, lo


def _dot(parts, y):
    hi, mid, lo = parts
    return _dot1(hi, y) + _dot1(mid, y) + _dot1(lo, y)


def _deint_body(a_ref, p_ref, n_ref, mel_ref, mer_ref, mol_ref, mor_ref,
                *outs):
    sels = [mel_ref[...], mer_ref[...], mol_ref[...], mor_ref[...]]
    zpad = jnp.zeros((8, 128), F32)
    for i, src in enumerate((a_ref, p_ref, n_ref)):
        parts = _split3(src[...])
        for j in range(4):
            dst = outs[4 * i + j]
            dst[pl.ds(0, 512), :] = _dot(parts, sels[j])
            dst[pl.ds(512, 8), :] = zpad


_DEINT = pl.pallas_call(
    _deint_body,
    out_shape=tuple(
        jax.ShapeDtypeStruct((520, 128), F32) for _ in range(12)
    ),
)


def _int_body(pel_r, per_r, pol_r, por_r, nel_r, ner_r, nol_r, nor_r,
              metl_ref, metr_ref, motl_ref, motr_ref,
              out_p_ref, out_n_ref):
    metl = metl_ref[...]
    metr = metr_ref[...]
    motl = motl_ref[...]
    motr = motr_ref[...]
    for el, er, ol, orr, dst in (
        (pel_r, per_r, pol_r, por_r, out_p_ref),
        (nel_r, ner_r, nol_r, nor_r, out_n_ref),
    ):
        dst[...] = (
            _dot(_split3(el[...]), metl) + _dot(_split3(er[...]), metr)
            + _dot(_split3(ol[...]), motl) + _dot(_split3(orr[...]), motr)
        )


_INT = pl.pallas_call(
    _int_body,
    out_shape=tuple(
        jax.ShapeDtypeStruct((512, 512), F32) for _ in range(2)
    ),
)


def kernel(anchor, positive, negative):
    planes = _DEINT(
        anchor, positive, negative,
        jnp.asarray(_ME[:, :128].copy(), BF16),
        jnp.asarray(_ME[:, 128:].copy(), BF16),
        jnp.asarray(_MO[:, :128].copy(), BF16),
        jnp.asarray(_MO[:, 128:].copy(), BF16),
    )
    outs = _POOL(*planes)
    return _INT(
        *outs,
        jnp.asarray(_ME.T[:128].copy(), BF16),
        jnp.asarray(_ME.T[128:].copy(), BF16),
        jnp.asarray(_MO.T[:128].copy(), BF16),
        jnp.asarray(_MO.T[128:].copy(), BF16),
    )


# speedup vs baseline: 1.3510x; 1.3510x over previous
import functools

import jax
import jax.numpy as jnp
import numpy as np
from jax import lax
from jax.experimental import pallas as pl
from jax.experimental.pallas import tpu as pltpu
from jax.experimental.pallas import tpu_sc as plsc

F32 = jnp.float32
_LOCS = (0, 16, 32, 48, 64, 80, 96, 111)


def _sc_pool():
    mesh = plsc.VectorSubcoreMesh(core_axis_name="c", subcore_axis_name="s")
    out_type = tuple(
        jax.ShapeDtypeStruct((512, 128), F32) for _ in range(8)
    )
    scratch = [pltpu.VMEM((24, 128), F32) for _ in range(12)] + [
        pltpu.VMEM((16, 128), F32) for _ in range(8)
    ] + [pltpu.SemaphoreType.DMA]

    @functools.partial(
        pl.kernel, out_type=out_type, mesh=mesh, scratch_types=scratch
    )
    def k(ael, aer, aol, aor, pel, per, pol, por, nel, ner, nol, nor,
          out_pel, out_per, out_pol, out_por,
          out_nel, out_ner, out_nol, out_nor,
          ael_v, aer_v, aol_v, aor_v, pel_v, per_v, pol_v, por_v,
          nel_v, ner_v, nol_v, nor_v,
          ebl_p, ebr_p, obl_p, obr_p, ebl_n, ebr_n, obl_n, obr_n, sem):
        wid = lax.axis_index("c") * 16 + lax.axis_index("s")
        r_top = 16 * wid
        ji = lax.iota(jnp.int32, 16)
        handles = [
            pltpu.async_copy(src.at[pl.ds(r_top, 24)], dst, sem)
            for src, dst in (
                (ael, ael_v), (aer, aer_v), (aol, aol_v), (aor, aor_v),
                (pel, pel_v), (per, per_v), (pol, pol_v), (por, por_v),
                (nel, nel_v), (ner, ner_v), (nol, nol_v), (nor, nor_v),
            )
        ]
        for h in handles:
            h.wait()

        def row_body(t, carry):
            lr = 2 * t
            for aeh, aoh, sel in (
                (ael_v, aol_v,
                 ((pel_v, pol_v, ebl_p, obl_p), (nel_v, nol_v, ebl_n, obl_n))),
                (aer_v, aor_v,
                 ((per_v, por_v, ebr_p, obr_p), (ner_v, nor_v, ebr_n, obr_n))),
            ):
                for l0 in _LOCS:
                    a_t = []
                    for r in range(3):
                        a_t.append(aeh[lr + r, pl.ds(l0, 16)])
                        a_t.append(aoh[lr + r, pl.ds(l0, 16)])
                        a_t.append(aeh[lr + r, pl.ds(l0 + 1, 16)])
                    for ceh, coh, eblk, oblk in sel:
                        c_t = []
                        for r in range(3):
                            c_t.append(ceh[lr + r, pl.ds(l0, 16)])
                            c_t.append(coh[lr + r, pl.ds(l0, 16)])
                            c_t.append(ceh[lr + r, pl.ds(l0 + 1, 16)])
                        is_min = eblk is ebl_p or eblk is ebr_p
                        bd = jnp.abs(a_t[0] - c_t[0])
                        bv = c_t[0]
                        for kk in range(1, 9):
                            dk = jnp.abs(a_t[kk] - c_t[kk])
                            m = (dk < bd) if is_min else (dk > bd)
                            bv = jnp.where(m, c_t[kk], bv)
                            bd = jnp.where(m, dk, bd)
                        eblk[lr, pl.ds(l0, 16)] = bv
                        eblk[lr + 1, pl.ds(l0, 16)] = bv
                        oblk[lr, pl.ds(l0, 16)] = bv
                        oblk[lr + 1, pl.ds(l0, 16)] = bv
            a127 = []
            p127 = []
            n127 = []
            for r in range(3):
                for taps, el_v, ol_v, er_v in (
                    (a127, ael_v, aol_v, aer_v),
                    (p127, pel_v, pol_v, per_v),
                    (n127, nel_v, nol_v, ner_v),
                ):
                    taps.append(el_v[lr + r, pl.ds(112, 16)][15])
                    taps.append(ol_v[lr + r, pl.ds(112, 16)][15])
                    taps.append(er_v[lr + r, pl.ds(0, 16)][0])
            for c127, is_min, eblk, oblk in (
                (p127, True, ebl_p, obl_p),
                (n127, False, ebl_n, obl_n),
            ):
                bd = jnp.abs(a127[0] - c127[0])
                bv = c127[0]
                for kk in range(1, 9):
                    dk = jnp.abs(a127[kk] - c127[kk])
                    m = (dk < bd) if is_min else (dk > bd)
                    bv = jnp.where(m, c127[kk], bv)
                    bd = jnp.where(m, dk, bd)
                for blk in (eblk, oblk):
                    for br in (lr, lr + 1):
                        v = blk[br, pl.ds(112, 16)]
                        blk[br, pl.ds(112, 16)] = jnp.where(ji == 15, bv, v)
            for coh, ebr, obr in ((por_v, ebr_p, obr_p), (nor_v, ebr_n, obr_n)):
                tail = ebr[lr, pl.ds(112, 16)]
                ev = jnp.where(ji == 15, tail[14], tail)
                ebr[lr, pl.ds(112, 16)] = ev
                ebr[lr + 1, pl.ds(112, 16)] = ev
                c0 = coh[lr, pl.ds(112, 16)]
                c1 = coh[lr + 1, pl.ds(112, 16)]
                obr[lr, pl.ds(112, 16)] = jnp.where(ji == 15, c0[15], tail)
                obr[lr + 1, pl.ds(112, 16)] = jnp.where(ji == 15, c1[15], tail)
            return carry

        lax.fori_loop(0, 8, row_body, 0)

        @pl.when(wid == 31)
        def _tail():
            for ceh, coh, celh, colh, ebl, obl, ebr, obr in (
                (per_v, por_v, pel_v, pol_v, ebl_p, obl_p, ebr_p, obr_p),
                (ner_v, nor_v, nel_v, nol_v, ebl_n, obl_n, ebr_n, obr_n),
            ):
                for blk, src15 in (
                    (ebl, celh), (obl, colh), (ebr, ceh), (obr, coh),
                ):
                    for tt in range(8):
                        blk[14, pl.ds(16 * tt, 16)] = blk[13, pl.ds(16 * tt, 16)]
                        blk[15, pl.ds(16 * tt, 16)] = src15[15, pl.ds(16 * tt, 16)]
                t14 = obr[14, pl.ds(112, 16)]
                c2 = coh[14, pl.ds(112, 16)]
                obr[14, pl.ds(112, 16)] = jnp.where(ji == 15, c2[15], t14)

        out_handles = [
            pltpu.async_copy(blk, out.at[pl.ds(r_top, 16)], sem)
            for blk, out in (
                (ebl_p, out_pel), (ebr_p, out_per), (obl_p, out_pol),
                (obr_p, out_por), (ebl_n, out_nel), (ebr_n, out_ner),
                (obl_n, out_nol), (obr_n, out_nor),
            )
        ]
        for h in out_handles:
            h.wait()

    return k


_POOL = _sc_pool()

_ME = np.zeros((512, 256), np.float32)
_ME[2 * np.arange(256), np.arange(256)] = 1.0
_MO = np.zeros((512, 256), np.float32)
_MO[2 * np.arange(256) + 1, np.arange(256)] = 1.0


BF16 = jnp.bfloat16


def _dot1(x, y):
    return lax.dot_general(
        x, y, (((1,), (0,)), ((), ())), preferred_element_type=F32,
    )


def _split3(x):
    hi = x.astype(BF16)
    r1 = x - hi.astype(F32)
    mid = r1.astype(BF16)
    lo = (r1 - mid.astype(F32)).astype(BF16)
    return hi, mid, lo


def _dot(parts, y):
    hi, mid, lo = parts
    return _dot1(hi, y) + _dot1(mid, y) + _dot1(lo, y)


def _deint_body(a_ref, p_ref, n_ref, mel_ref, mer_ref, mol_ref, mor_ref,
                *outs):
    sels = [mel_ref[...], mer_ref[...], mol_ref[...], mor_ref[...]]
    zpad = jnp.zeros((8, 128), F32)
    for i, src in enumerate((a_ref, p_ref, n_ref)):
        parts = _split3(src[...])
        for j in range(4):
            dst = outs[4 * i + j]
            dst[pl.ds(0, 512), :] = _dot(parts, sels[j])
            dst[pl.ds(512, 8), :] = zpad


_DEINT = pl.pallas_call(
    _deint_body,
    out_shape=tuple(
        jax.ShapeDtypeStruct((520, 128), F32) for _ in range(12)
    ),
)


def _int_body(pel_r, per_r, pol_r, por_r, nel_r, ner_r, nol_r, nor_r,
              metl_ref, metr_ref, motl_ref, motr_ref,
              out_p_ref, out_n_ref):
    metl = metl_ref[...]
    metr = metr_ref[...]
    motl = motl_ref[...]
    motr = motr_ref[...]
    for el, er, ol, orr, dst in (
        (pel_r, per_r, pol_r, por_r, out_p_ref),
        (nel_r, ner_r, nol_r, nor_r, out_n_ref),
    ):
        dst[...] = (
            _dot(_split3(el[...]), metl) + _dot(_split3(er[...]), metr)
            + _dot(_split3(ol[...]), motl) + _dot(_split3(orr[...]), motr)
        )


_INT = pl.pallas_call(
    _int_body,
    out_shape=tuple(
        jax.ShapeDtypeStruct((512, 512), F32) for _ in range(2)
    ),
)


def kernel(anchor, positive, negative):
    planes = _DEINT(
        anchor, positive, negative,
        jnp.asarray(_ME[:, :128].copy(), BF16),
        jnp.asarray(_ME[:, 128:].copy(), BF16),
        jnp.asarray(_MO[:, :128].copy(), BF16),
        jnp.asarray(_MO[:, 128:].copy(), BF16),
    )
    outs = _POOL(*planes)
    return _INT(
        *outs,
        jnp.asarray(_ME.T[:128].copy(), BF16),
        jnp.asarray(_ME.T[128:].copy(), BF16),
        jnp.asarray(_MO.T[:128].copy(), BF16),
        jnp.asarray(_MO.T[128:].copy(), BF16),
    )
